# baseline (speedup 1.0000x reference)
import functools

import jax
import jax.numpy as jnp
from jax import lax
from jax.experimental import pallas as pl
from jax.experimental.pallas import tpu as pltpu
from jax.experimental.pallas import tpu_sc as plsc

L = 16
NC, NS = 2, 16
NW = NC * NS
SUB = 513
CH = L * SUB


def _sc_partials(nnz: int, in_dims: int, out_dims: int):
    n_full = nnz // CH
    tail = nnz - n_full * CH
    n_chunks = n_full + (1 if tail else 0)
    k_max = -(-n_chunks // NW)

    mesh = plsc.VectorSubcoreMesh(
        core_axis_name="c", subcore_axis_name="s",
        num_cores=NC, num_subcores=NS)

    @functools.partial(
        pl.kernel,
        out_type=jax.ShapeDtypeStruct((NW, out_dims), jnp.float32),
        mesh=mesh,
        compiler_params=pltpu.CompilerParams(needs_layout_passes=False),
        scratch_types=[
            pltpu.VMEM((in_dims,), jnp.float32),
            pltpu.VMEM((CH,), jnp.int32),
            pltpu.VMEM((CH,), jnp.int32),
            pltpu.VMEM((CH,), jnp.int32),
            pltpu.VMEM((CH,), jnp.int32),
            pltpu.VMEM((CH,), jnp.int32),
            pltpu.VMEM((CH,), jnp.int32),
            pltpu.VMEM((CH,), jnp.float32),
            pltpu.VMEM((CH,), jnp.float32),
            pltpu.VMEM((CH,), jnp.float32),
            pltpu.VMEM((out_dims,), jnp.float32),
            pltpu.SemaphoreType.DMA,
            pltpu.SemaphoreType.DMA,
            pltpu.SemaphoreType.DMA,
            pltpu.SemaphoreType.DMA,
        ],
    )
    def sc_kernel(x_hbm, rows_hbm, cols_hbm, vals_hbm, part_hbm,
                  x_v, rows_v0, rows_v1, rows_v2, cols_v0, cols_v1, cols_v2,
                  vals_v0, vals_v1, vals_v2, acc_v, sem0, sem1, sem2, xsem):
        wid = lax.axis_index("s") * NC + lax.axis_index("c")
        sems = (sem0, sem1, sem2)
        rows_b = (rows_v0, rows_v1, rows_v2)
        cols_b = (cols_v0, cols_v1, cols_v2)
        vals_b = (vals_v0, vals_v1, vals_v2)

        def chunk_copies(cid_val, buf, n_elems):
            base = pl.multiple_of(cid_val * CH, 8)
            s = sems[buf]
            return (
                pltpu.make_async_copy(rows_hbm.at[pl.ds(base, n_elems)],
                                      rows_b[buf].at[pl.ds(0, n_elems)], s),
                pltpu.make_async_copy(cols_hbm.at[pl.ds(base, n_elems)],
                                      cols_b[buf].at[pl.ds(0, n_elems)], s),
                pltpu.make_async_copy(vals_hbm.at[pl.ds(base, n_elems)],
                                      vals_b[buf].at[pl.ds(0, n_elems)], s),
            )

        def issue(k, buf):
            cid = k * NW + wid

            @pl.when(cid < n_full)
            def _():
                for c in chunk_copies(cid, buf, CH):
                    c.start()

            if tail and (k + 1) * NW > n_full:
                @pl.when(cid == n_full)
                def _():
                    for c in chunk_copies(cid, buf, tail):
                        c.start()

        def wait(k, buf):
            cid = k * NW + wid

            @pl.when(cid < n_full)
            def _():
                for c in chunk_copies(cid, buf, CH):
                    c.wait()

            if tail and (k + 1) * NW > n_full:
                @pl.when(cid == n_full)
                def _():
                    for c in chunk_copies(cid, buf, tail):
                        c.wait()

        issue(0, 0)
        if k_max > 1:
            issue(1, 1)
        xcopy = pltpu.make_async_copy(x_hbm, x_v, xsem)
        xcopy.start()

        def zero_step(i, _):
            acc_v[pl.ds(pl.multiple_of(i * L, L), L)] = jnp.zeros((L,), jnp.float32)
            return 0
        lax.fori_loop(0, out_dims // L, zero_step, 0)
        xcopy.wait()

        lane = lax.broadcasted_iota(jnp.int32, (L,), 0)
        lane_base = lane * SUB

        def proc_sub(buf):
            rv = rows_b[buf]
            cv = cols_b[buf]
            vv = vals_b[buf]
            r0 = plsc.load_gather(rv, [lane_base])

            @plsc.parallel_loop(0, SUB, 1, unroll=8,
                                carry=(r0, jnp.zeros((L,), jnp.float32)))
            def step(i, carry):
                r_prev, acc = carry
                idx = lane_base + i
                r = plsc.load_gather(rv, [idx])
                c = plsc.load_gather(cv, [idx])
                v = plsc.load_gather(vv, [idx])
                xg = plsc.load_gather(x_v, [c])
                p = v * xg
                changed = r != r_prev
                plsc.addupdate_scatter(acc_v, [r_prev], acc, mask=changed)
                acc = jnp.where(changed, p, acc + p)
                return r, acc

            r_last, acc = step
            plsc.addupdate_scatter(acc_v, [r_last], acc)

        def proc_linear(buf, n_elems):
            rv = rows_b[buf]
            cv = cols_b[buf]
            vv = vals_b[buf]

            def step(i, _):
                off = pl.multiple_of(i * L, L)
                c = cv[pl.ds(off, L)]
                r = rv[pl.ds(off, L)]
                v = vv[pl.ds(off, L)]
                xg = plsc.load_gather(x_v, [c])
                plsc.addupdate_scatter(acc_v, [r], v * xg)
                return 0
            lax.fori_loop(0, n_elems // L, step, 0)
            rem = n_elems % L
            if rem:
                base = n_elems - rem
                m = lane < rem
                c = jnp.where(m, cv[pl.ds(base, L)], 0)
                r = jnp.where(m, rv[pl.ds(base, L)], 0)
                v = jnp.where(m, vv[pl.ds(base, L)], 0.0)
                xg = plsc.load_gather(x_v, [c], mask=m)
                plsc.addupdate_scatter(acc_v, [r], v * xg, mask=m)

        for k in range(k_max):
            buf = k % 3
            if k + 2 < k_max:
                issue(k + 2, (k + 2) % 3)
            wait(k, buf)
            cid = k * NW + wid

            @pl.when(cid < n_full)
            def _():
                proc_sub(buf)

            if tail and (k + 1) * NW > n_full:
                @pl.when(cid == n_full)
                def _():
                    proc_linear(buf, tail)

        pltpu.sync_copy(acc_v, part_hbm.at[wid])

    return sc_kernel


def _tc_reduce(part, b2):
    n, d = part.shape
    blk = d // 8

    def body(p_ref, b_ref, o_ref):
        o_ref[...] = jnp.sum(p_ref[...], axis=0, keepdims=True) + b_ref[...]

    return pl.pallas_call(
        body,
        grid=(8,),
        in_specs=[pl.BlockSpec((n, blk), lambda i: (0, i)),
                  pl.BlockSpec((1, blk), lambda i: (0, i))],
        out_specs=pl.BlockSpec((1, blk), lambda i: (0, i)),
        out_shape=jax.ShapeDtypeStruct((1, d), jnp.float32),
    )(part, b2)


def kernel(x, rows, cols, vals, B):
    nnz = rows.shape[0]
    in_dims = x.shape[0]
    out_dims = B.shape[0]
    part = _sc_partials(nnz, in_dims, out_dims)(x, rows, cols, vals)
    out = _tc_reduce(part, B.reshape(1, out_dims))
    return out.reshape(out_dims)


# speedup vs baseline: 1.0592x; 1.0592x over previous
import functools

import jax
import jax.numpy as jnp
from jax import lax
from jax.experimental import pallas as pl
from jax.experimental.pallas import tpu as pltpu
from jax.experimental.pallas import tpu_sc as plsc

L = 16
NC, NS = 2, 16
NW = NC * NS
SUB = 513
CH = L * SUB


def _sc_partials(nnz: int, in_dims: int, out_dims: int):
    n_full = nnz // CH
    tail = nnz - n_full * CH
    n_chunks = n_full + (1 if tail else 0)
    k_max = -(-n_chunks // NW)

    mesh = plsc.VectorSubcoreMesh(
        core_axis_name="c", subcore_axis_name="s",
        num_cores=NC, num_subcores=NS)

    @functools.partial(
        pl.kernel,
        out_type=jax.ShapeDtypeStruct((NW, out_dims), jnp.float32),
        mesh=mesh,
        compiler_params=pltpu.CompilerParams(needs_layout_passes=False),
        scratch_types=[
            pltpu.VMEM((in_dims,), jnp.float32),
            pltpu.VMEM((CH,), jnp.int32),
            pltpu.VMEM((CH,), jnp.int32),
            pltpu.VMEM((CH,), jnp.int32),
            pltpu.VMEM((CH,), jnp.int32),
            pltpu.VMEM((CH,), jnp.int32),
            pltpu.VMEM((CH,), jnp.int32),
            pltpu.VMEM((CH,), jnp.float32),
            pltpu.VMEM((CH,), jnp.float32),
            pltpu.VMEM((CH,), jnp.float32),
            pltpu.VMEM((out_dims,), jnp.float32),
            pltpu.SemaphoreType.DMA,
            pltpu.SemaphoreType.DMA,
            pltpu.SemaphoreType.DMA,
            pltpu.SemaphoreType.DMA,
        ],
    )
    def sc_kernel(x_hbm, rows_hbm, cols_hbm, vals_hbm, part_hbm,
                  x_v, rows_v0, rows_v1, rows_v2, cols_v0, cols_v1, cols_v2,
                  vals_v0, vals_v1, vals_v2, acc_v, sem0, sem1, sem2, xsem):
        wid = lax.axis_index("s") * NC + lax.axis_index("c")
        sems = (sem0, sem1, sem2)
        rows_b = (rows_v0, rows_v1, rows_v2)
        cols_b = (cols_v0, cols_v1, cols_v2)
        vals_b = (vals_v0, vals_v1, vals_v2)

        def chunk_copies(cid_val, buf, n_elems):
            base = pl.multiple_of(cid_val * CH, 8)
            s = sems[buf]
            return (
                pltpu.make_async_copy(rows_hbm.at[pl.ds(base, n_elems)],
                                      rows_b[buf].at[pl.ds(0, n_elems)], s),
                pltpu.make_async_copy(cols_hbm.at[pl.ds(base, n_elems)],
                                      cols_b[buf].at[pl.ds(0, n_elems)], s),
                pltpu.make_async_copy(vals_hbm.at[pl.ds(base, n_elems)],
                                      vals_b[buf].at[pl.ds(0, n_elems)], s),
            )

        def issue(k, buf):
            cid = k * NW + wid

            @pl.when(cid < n_full)
            def _():
                for c in chunk_copies(cid, buf, CH):
                    c.start()

            if tail and (k + 1) * NW > n_full:
                @pl.when(cid == n_full)
                def _():
                    for c in chunk_copies(cid, buf, tail):
                        c.start()

        def wait(k, buf):
            cid = k * NW + wid

            @pl.when(cid < n_full)
            def _():
                for c in chunk_copies(cid, buf, CH):
                    c.wait()

            if tail and (k + 1) * NW > n_full:
                @pl.when(cid == n_full)
                def _():
                    for c in chunk_copies(cid, buf, tail):
                        c.wait()

        issue(0, 0)
        if k_max > 1:
            issue(1, 1)
        xcopy = pltpu.make_async_copy(x_hbm, x_v, xsem)
        xcopy.start()

        def zero_step(i, _):
            acc_v[pl.ds(pl.multiple_of(i * L, L), L)] = jnp.zeros((L,), jnp.float32)
            return 0
        lax.fori_loop(0, out_dims // L, zero_step, 0)
        xcopy.wait()

        lane = lax.broadcasted_iota(jnp.int32, (L,), 0)
        lane_base = lane * SUB

        def proc_sub(buf):
            rv = rows_b[buf]
            cv = cols_b[buf]
            vv = vals_b[buf]
            r0 = plsc.load_gather(rv, [lane_base])

            @plsc.parallel_loop(0, SUB, 1, unroll=8,
                                carry=(r0, jnp.zeros((L,), jnp.float32)))
            def step(i, carry):
                r_prev, acc = carry
                idx = lane_base + i
                r = plsc.load_gather(rv, [idx])
                c = plsc.load_gather(cv, [idx])
                v = plsc.load_gather(vv, [idx])
                xg = plsc.load_gather(x_v, [c])
                p = v * xg
                changed = r != r_prev
                plsc.addupdate_scatter(acc_v, [r_prev], acc, mask=changed)
                acc = jnp.where(changed, p, acc + p)
                return r, acc

            r_last, acc = step
            plsc.addupdate_scatter(acc_v, [r_last], acc)

        def proc_linear(buf, n_elems):
            rv = rows_b[buf]
            cv = cols_b[buf]
            vv = vals_b[buf]

            def step(i, _):
                off = pl.multiple_of(i * L, L)
                c = cv[pl.ds(off, L)]
                r = rv[pl.ds(off, L)]
                v = vv[pl.ds(off, L)]
                xg = plsc.load_gather(x_v, [c])
                plsc.addupdate_scatter(acc_v, [r], v * xg)
                return 0
            lax.fori_loop(0, n_elems // L, step, 0)
            rem = n_elems % L
            if rem:
                base = n_elems - rem
                m = lane < rem
                c = jnp.where(m, cv[pl.ds(base, L)], 0)
                r = jnp.where(m, rv[pl.ds(base, L)], 0)
                v = jnp.where(m, vv[pl.ds(base, L)], 0.0)
                xg = plsc.load_gather(x_v, [c], mask=m)
                plsc.addupdate_scatter(acc_v, [r], v * xg, mask=m)

        for k in range(k_max):
            buf = k % 3
            if k + 2 < k_max:
                issue(k + 2, (k + 2) % 3)
            wait(k, buf)
            cid = k * NW + wid

            @pl.when(cid < n_full)
            def _():
                proc_sub(buf)

            if tail and (k + 1) * NW > n_full:
                @pl.when(cid == n_full)
                def _():
                    proc_linear(buf, tail)

        pltpu.sync_copy(acc_v, part_hbm.at[wid])

    return sc_kernel


def _tc_reduce(part, b2):
    def body(p_ref, b_ref, o_ref):
        o_ref[...] = jnp.sum(p_ref[...], axis=0, keepdims=True) + b_ref[...]

    return pl.pallas_call(
        body,
        out_shape=jax.ShapeDtypeStruct((1, part.shape[1]), jnp.float32),
    )(part, b2)


def kernel(x, rows, cols, vals, B):
    nnz = rows.shape[0]
    in_dims = x.shape[0]
    out_dims = B.shape[0]
    part = _sc_partials(nnz, in_dims, out_dims)(x, rows, cols, vals)
    out = _tc_reduce(part, B.reshape(1, out_dims))
    return out.reshape(out_dims)


# speedup vs baseline: 1.0892x; 1.0284x over previous
import functools

import jax
import jax.numpy as jnp
from jax import lax
from jax.experimental import pallas as pl
from jax.experimental.pallas import tpu as pltpu
from jax.experimental.pallas import tpu_sc as plsc

L = 16
NC, NS = 2, 16
NW = NC * NS
SUB = 513
CH = L * SUB


def _sc_partials(nnz: int, in_dims: int, out_dims: int):
    n_full = nnz // CH
    tail = nnz - n_full * CH
    n_chunks = n_full + (1 if tail else 0)
    k_max = -(-n_chunks // NW)

    mesh = plsc.VectorSubcoreMesh(
        core_axis_name="c", subcore_axis_name="s",
        num_cores=NC, num_subcores=NS)

    @functools.partial(
        pl.kernel,
        out_type=jax.ShapeDtypeStruct((NW, out_dims), jnp.float32),
        mesh=mesh,
        compiler_params=pltpu.CompilerParams(needs_layout_passes=False),
        scratch_types=[
            pltpu.VMEM((in_dims,), jnp.float32),
            pltpu.VMEM((CH,), jnp.int32),
            pltpu.VMEM((CH,), jnp.int32),
            pltpu.VMEM((CH,), jnp.int32),
            pltpu.VMEM((CH,), jnp.int32),
            pltpu.VMEM((CH,), jnp.float32),
            pltpu.VMEM((CH,), jnp.float32),
            pltpu.VMEM((out_dims,), jnp.float32),
            pltpu.SemaphoreType.DMA,
            pltpu.SemaphoreType.DMA,
        ],
    )
    def sc_kernel(x_hbm, rows_hbm, cols_hbm, vals_hbm, part_hbm,
                  x_v, rows_v0, rows_v1, cols_v0, cols_v1, vals_v0, vals_v1,
                  acc_v, sem0, sem1):
        wid = lax.axis_index("s") * NC + lax.axis_index("c")
        sems = (sem0, sem1)
        rows_b = (rows_v0, rows_v1)
        cols_b = (cols_v0, cols_v1)
        vals_b = (vals_v0, vals_v1)

        def chunk_copies(cid_val, buf, n_elems):
            base = pl.multiple_of(cid_val * CH, 8)
            s = sems[buf]
            return (
                pltpu.make_async_copy(rows_hbm.at[pl.ds(base, n_elems)],
                                      rows_b[buf].at[pl.ds(0, n_elems)], s),
                pltpu.make_async_copy(cols_hbm.at[pl.ds(base, n_elems)],
                                      cols_b[buf].at[pl.ds(0, n_elems)], s),
                pltpu.make_async_copy(vals_hbm.at[pl.ds(base, n_elems)],
                                      vals_b[buf].at[pl.ds(0, n_elems)], s),
            )

        def issue(k, buf):
            cid = k * NW + wid

            @pl.when(cid < n_full)
            def _():
                for c in chunk_copies(cid, buf, CH):
                    c.start()

            if tail and (k + 1) * NW > n_full:
                @pl.when(cid == n_full)
                def _():
                    for c in chunk_copies(cid, buf, tail):
                        c.start()

        def wait(k, buf):
            cid = k * NW + wid

            @pl.when(cid < n_full)
            def _():
                for c in chunk_copies(cid, buf, CH):
                    c.wait()

            if tail and (k + 1) * NW > n_full:
                @pl.when(cid == n_full)
                def _():
                    for c in chunk_copies(cid, buf, tail):
                        c.wait()

        issue(0, 0)
        xcopy = pltpu.make_async_copy(x_hbm, x_v, sems[1])
        xcopy.start()

        def zero_step(i, _):
            acc_v[pl.ds(pl.multiple_of(i * L, L), L)] = jnp.zeros((L,), jnp.float32)
            return 0
        lax.fori_loop(0, out_dims // L, zero_step, 0)
        xcopy.wait()

        lane = lax.broadcasted_iota(jnp.int32, (L,), 0)
        lane_base = lane * SUB

        def proc_sub(buf):
            rv = rows_b[buf]
            cv = cols_b[buf]
            vv = vals_b[buf]
            r0 = plsc.load_gather(rv, [lane_base])

            @plsc.parallel_loop(0, SUB, 1, unroll=4,
                                carry=(r0, jnp.zeros((L,), jnp.float32)))
            def step(i, carry):
                r_prev, acc = carry
                idx = lane_base + i
                r = plsc.load_gather(rv, [idx])
                c = plsc.load_gather(cv, [idx])
                v = plsc.load_gather(vv, [idx])
                xg = plsc.load_gather(x_v, [c])
                p = v * xg
                changed = r != r_prev
                plsc.addupdate_scatter(acc_v, [r_prev], acc, mask=changed)
                acc = jnp.where(changed, p, acc + p)
                return r, acc

            r_last, acc = step
            plsc.addupdate_scatter(acc_v, [r_last], acc)

        def proc_linear(buf, n_elems):
            rv = rows_b[buf]
            cv = cols_b[buf]
            vv = vals_b[buf]

            def step(i, _):
                off = pl.multiple_of(i * L, L)
                c = cv[pl.ds(off, L)]
                r = rv[pl.ds(off, L)]
                v = vv[pl.ds(off, L)]
                xg = plsc.load_gather(x_v, [c])
                plsc.addupdate_scatter(acc_v, [r], v * xg)
                return 0
            lax.fori_loop(0, n_elems // L, step, 0)
            rem = n_elems % L
            if rem:
                base = n_elems - rem
                m = lane < rem
                c = jnp.where(m, cv[pl.ds(base, L)], 0)
                r = jnp.where(m, rv[pl.ds(base, L)], 0)
                v = jnp.where(m, vv[pl.ds(base, L)], 0.0)
                xg = plsc.load_gather(x_v, [c], mask=m)
                plsc.addupdate_scatter(acc_v, [r], v * xg, mask=m)

        for k in range(k_max):
            buf = k & 1
            if k + 1 < k_max:
                issue(k + 1, buf ^ 1)
            wait(k, buf)
            cid = k * NW + wid

            @pl.when(cid < n_full)
            def _():
                proc_sub(buf)

            if tail and (k + 1) * NW > n_full:
                @pl.when(cid == n_full)
                def _():
                    proc_linear(buf, tail)

        pltpu.sync_copy(acc_v, part_hbm.at[wid])

    return sc_kernel


def _tc_reduce(part, b2):
    def body(p_ref, b_ref, o_ref):
        o_ref[...] = jnp.sum(p_ref[...], axis=0, keepdims=True) + b_ref[...]

    return pl.pallas_call(
        body,
        out_shape=jax.ShapeDtypeStruct((1, part.shape[1]), jnp.float32),
    )(part, b2)


def kernel(x, rows, cols, vals, B):
    nnz = rows.shape[0]
    in_dims = x.shape[0]
    out_dims = B.shape[0]
    part = _sc_partials(nnz, in_dims, out_dims)(x, rows, cols, vals)
    out = _tc_reduce(part, B.reshape(1, out_dims))
    return out.reshape(out_dims)


# speedup vs baseline: 1.1060x; 1.0154x over previous
import functools

import jax
import jax.numpy as jnp
from jax import lax
from jax.experimental import pallas as pl
from jax.experimental.pallas import tpu as pltpu
from jax.experimental.pallas import tpu_sc as plsc

L = 16
NC, NS = 2, 16
NW = NC * NS
SUB = 513
CH = L * SUB


def _sc_partials(nnz: int, in_dims: int, out_dims: int):
    n_full = nnz // CH
    tail = nnz - n_full * CH
    n_chunks = n_full + (1 if tail else 0)
    k_max = -(-n_chunks // NW)

    mesh = plsc.VectorSubcoreMesh(
        core_axis_name="c", subcore_axis_name="s",
        num_cores=NC, num_subcores=NS)

    @functools.partial(
        pl.kernel,
        out_type=jax.ShapeDtypeStruct((NW, out_dims), jnp.float32),
        mesh=mesh,
        compiler_params=pltpu.CompilerParams(needs_layout_passes=False),
        scratch_types=[
            pltpu.VMEM((in_dims,), jnp.float32),
            pltpu.VMEM((CH,), jnp.int32),
            pltpu.VMEM((CH,), jnp.int32),
            pltpu.VMEM((CH,), jnp.int32),
            pltpu.VMEM((CH,), jnp.int32),
            pltpu.VMEM((CH,), jnp.float32),
            pltpu.VMEM((CH,), jnp.float32),
            pltpu.VMEM((out_dims,), jnp.float32),
            pltpu.SemaphoreType.DMA,
            pltpu.SemaphoreType.DMA,
        ],
    )
    def sc_kernel(x_hbm, rows_hbm, cols_hbm, vals_hbm, part_hbm,
                  x_v, rows_v0, rows_v1, cols_v0, cols_v1, vals_v0, vals_v1,
                  acc_v, sem0, sem1):
        wid = lax.axis_index("s") * NC + lax.axis_index("c")
        sems = (sem0, sem1)
        rows_b = (rows_v0, rows_v1)
        cols_b = (cols_v0, cols_v1)
        vals_b = (vals_v0, vals_v1)

        def chunk_copies(cid_val, buf, n_elems):
            base = pl.multiple_of(cid_val * CH, 8)
            s = sems[buf]
            return (
                pltpu.make_async_copy(rows_hbm.at[pl.ds(base, n_elems)],
                                      rows_b[buf].at[pl.ds(0, n_elems)], s),
                pltpu.make_async_copy(cols_hbm.at[pl.ds(base, n_elems)],
                                      cols_b[buf].at[pl.ds(0, n_elems)], s),
                pltpu.make_async_copy(vals_hbm.at[pl.ds(base, n_elems)],
                                      vals_b[buf].at[pl.ds(0, n_elems)], s),
            )

        def issue(k, buf):
            cid = k * NW + wid

            @pl.when(cid < n_full)
            def _():
                for c in chunk_copies(cid, buf, CH):
                    c.start()

            if tail and (k + 1) * NW > n_full:
                @pl.when(cid == n_full)
                def _():
                    for c in chunk_copies(cid, buf, tail):
                        c.start()

        def wait(k, buf):
            cid = k * NW + wid

            @pl.when(cid < n_full)
            def _():
                for c in chunk_copies(cid, buf, CH):
                    c.wait()

            if tail and (k + 1) * NW > n_full:
                @pl.when(cid == n_full)
                def _():
                    for c in chunk_copies(cid, buf, tail):
                        c.wait()

        issue(0, 0)
        xcopy = pltpu.make_async_copy(x_hbm, x_v, sems[1])
        xcopy.start()

        def zero_step(i, _):
            acc_v[pl.ds(pl.multiple_of(i * L, L), L)] = jnp.zeros((L,), jnp.float32)
            return 0
        lax.fori_loop(0, out_dims // L, zero_step, 0)
        xcopy.wait()

        lane = lax.broadcasted_iota(jnp.int32, (L,), 0)
        lane_base = lane * SUB

        def proc_sub(buf):
            rv = rows_b[buf]
            cv = cols_b[buf]
            vv = vals_b[buf]
            r0 = plsc.load_gather(rv, [lane_base])

            @plsc.parallel_loop(0, SUB, 1, unroll=2,
                                carry=(r0, jnp.zeros((L,), jnp.float32)))
            def step(i, carry):
                r_prev, acc = carry
                idx = lane_base + i
                r = plsc.load_gather(rv, [idx])
                c = plsc.load_gather(cv, [idx])
                v = plsc.load_gather(vv, [idx])
                xg = plsc.load_gather(x_v, [c])
                p = v * xg
                changed = r != r_prev
                plsc.addupdate_scatter(acc_v, [r_prev], acc, mask=changed)
                acc = jnp.where(changed, p, acc + p)
                return r, acc

            r_last, acc = step
            plsc.addupdate_scatter(acc_v, [r_last], acc)

        def proc_linear(buf, n_elems):
            rv = rows_b[buf]
            cv = cols_b[buf]
            vv = vals_b[buf]

            def step(i, _):
                off = pl.multiple_of(i * L, L)
                c = cv[pl.ds(off, L)]
                r = rv[pl.ds(off, L)]
                v = vv[pl.ds(off, L)]
                xg = plsc.load_gather(x_v, [c])
                plsc.addupdate_scatter(acc_v, [r], v * xg)
                return 0
            lax.fori_loop(0, n_elems // L, step, 0)
            rem = n_elems % L
            if rem:
                base = n_elems - rem
                m = lane < rem
                c = jnp.where(m, cv[pl.ds(base, L)], 0)
                r = jnp.where(m, rv[pl.ds(base, L)], 0)
                v = jnp.where(m, vv[pl.ds(base, L)], 0.0)
                xg = plsc.load_gather(x_v, [c], mask=m)
                plsc.addupdate_scatter(acc_v, [r], v * xg, mask=m)

        for k in range(k_max):
            buf = k & 1
            if k + 1 < k_max:
                issue(k + 1, buf ^ 1)
            wait(k, buf)
            cid = k * NW + wid

            @pl.when(cid < n_full)
            def _():
                proc_sub(buf)

            if tail and (k + 1) * NW > n_full:
                @pl.when(cid == n_full)
                def _():
                    proc_linear(buf, tail)

        pltpu.sync_copy(acc_v, part_hbm.at[wid])

    return sc_kernel


def _tc_reduce(part, b2):
    def body(p_ref, b_ref, o_ref):
        o_ref[...] = jnp.sum(p_ref[...], axis=0, keepdims=True) + b_ref[...]

    return pl.pallas_call(
        body,
        out_shape=jax.ShapeDtypeStruct((1, part.shape[1]), jnp.float32),
    )(part, b2)


def kernel(x, rows, cols, vals, B):
    nnz = rows.shape[0]
    in_dims = x.shape[0]
    out_dims = B.shape[0]
    part = _sc_partials(nnz, in_dims, out_dims)(x, rows, cols, vals)
    out = _tc_reduce(part, B.reshape(1, out_dims))
    return out.reshape(out_dims)
